# Initial kernel scaffold; baseline (speedup 1.0000x reference)
import jax
import jax.numpy as jnp
from jax.experimental import pallas as pl


def kernel(heatmap, reg, height, dim, rot, vel):
    raise NotImplementedError("write your pallas kernel here")


# speedup vs baseline: 6.4092x; 6.4092x over previous
import jax
import jax.numpy as jnp
from jax import lax
from jax.experimental import pallas as pl
from jax.experimental.pallas import tpu as pltpu

_K = 500
_OUT_SIZE_FACTOR = 4.0
_SCORE_THRESHOLD = 0.1
_GRID = 2048.0


def _suppress_body(x_ref, o_ref):
    x = x_ref[0]
    s = 1.0 / (1.0 + jnp.exp(-x))
    ninf = jnp.full((1, s.shape[1]), -jnp.inf, s.dtype)
    up = jnp.concatenate([s[1:], ninf], axis=0)
    dn = jnp.concatenate([ninf, s[:-1]], axis=0)
    r = jnp.maximum(jnp.maximum(s, up), dn)
    ninfc = jnp.full((r.shape[0], 1), -jnp.inf, r.dtype)
    lt = jnp.concatenate([r[:, 1:], ninfc], axis=1)
    rt = jnp.concatenate([ninfc, r[:, :-1]], axis=1)
    m = jnp.maximum(jnp.maximum(r, lt), rt)
    o_ref[0] = jnp.where(s == m, s, 0.0)


def _suppress(heatmap):
    B, C, H, W = heatmap.shape
    hm = heatmap.reshape(B * C, H, W)
    out = pl.pallas_call(
        _suppress_body,
        grid=(B * C,),
        in_specs=[pl.BlockSpec((1, H, W), lambda i: (i, 0, 0))],
        out_specs=pl.BlockSpec((1, H, W), lambda i: (i, 0, 0)),
        out_shape=jax.ShapeDtypeStruct((B * C, H, W), jnp.float32),
    )(hm)
    return out.reshape(B, C * H * W)


def kernel(heatmap, reg, height, dim, rot, vel):
    B, C, H, W = heatmap.shape
    HW = H * W
    sup = _suppress(heatmap)
    scores, ind = lax.top_k(sup, _K)
    cls = (ind // HW).astype(jnp.int32)
    pix = ind % HW
    ys = jnp.floor(pix.astype(jnp.float32) / float(W))
    xs = (pix % W).astype(jnp.float32)

    bbox = jnp.concatenate([reg, height, dim, rot, vel], axis=1)
    D = bbox.shape[1]
    bbox = bbox.reshape(B, D, HW)
    vals = jnp.take_along_axis(bbox, jnp.broadcast_to(pix[:, None, :], (B, D, _K)), axis=2)
    x = (xs + vals[:, 0, :]) * _OUT_SIZE_FACTOR
    y = (ys + vals[:, 1, :]) * _OUT_SIZE_FACTOR
    hei = vals[:, 2, :]
    dims = jnp.exp(vals[:, 3:6, :])
    rot_a = jnp.arctan2(vals[:, 6, :], vals[:, 7, :])
    mask = (scores > _SCORE_THRESHOLD) & (x > 0) & (x < _GRID) & (y > 0) & (y < _GRID)
    scores_m = jnp.where(mask, scores, 0.0)
    out = jnp.stack([x, y, hei, dims[:, 0], dims[:, 1], dims[:, 2],
                     rot_a, vals[:, 8, :], vals[:, 9, :],
                     scores_m, cls.astype(jnp.float32)], axis=2)
    return out


# speedup vs baseline: 25.1674x; 3.9268x over previous
import functools

import jax
import jax.numpy as jnp
from jax import lax
from jax.experimental import pallas as pl
from jax.experimental.pallas import tpu as pltpu
from jax.experimental.pallas import tpu_sc as plsc

_K = 500
_OUT_SIZE_FACTOR = 4.0
_SCORE_THRESHOLD = 0.1
_GRIDB = 2048.0

_B, _C, _H, _W = 4, 2, 512, 512
_HW = _H * _W
_CHW = _C * _HW
_NSLICE = 8
_SLICE = _CHW // _NSLICE
_NBUCK = 512
_CAND_CAP = 2048
_MERGE_CAP = 4096
_OROWS = 512

_PI = 3.14159265358979
_PI_2 = 1.5707963267948966

_ATAN_C = (0.9999999981419218, -0.33333292787705715, 0.19998532263347163,
           -0.14264888583256646, 0.10958341227667072, -0.08427560725997432,
           0.05845650556360228, -0.0317490822377156, 0.011256772475624163,
           -0.001877352082647006)


def _suppress_body(x_ref, o_ref):
    x = x_ref[0]
    s = 1.0 / (1.0 + jnp.exp(-x))
    ninf = jnp.full((1, s.shape[1]), -jnp.inf, s.dtype)
    up = jnp.concatenate([s[1:], ninf], axis=0)
    dn = jnp.concatenate([ninf, s[:-1]], axis=0)
    r = jnp.maximum(jnp.maximum(s, up), dn)
    ninfc = jnp.full((r.shape[0], 1), -jnp.inf, r.dtype)
    lt = jnp.concatenate([r[:, 1:], ninfc], axis=1)
    rt = jnp.concatenate([ninfc, r[:, :-1]], axis=1)
    m = jnp.maximum(jnp.maximum(r, lt), rt)
    o_ref[0] = jnp.where(s == m, s, 0.0)


def _suppress(heatmap):
    B, C, H, W = heatmap.shape
    hm = heatmap.reshape(B * C, H, W)
    out = pl.pallas_call(
        _suppress_body,
        grid=(B * C,),
        in_specs=[pl.BlockSpec((1, H, W), lambda i: (i, 0, 0))],
        out_specs=pl.BlockSpec((1, H, W), lambda i: (i, 0, 0)),
        out_shape=jax.ShapeDtypeStruct((B * C, H, W), jnp.float32),
    )(hm)
    return out.reshape(B * C * H * W)


def _bucket_of(v):
    t = 2.0 - jnp.maximum(v, 1e-6)
    u = lax.bitcast_convert_type(t, jnp.int32)
    return lax.shift_right_logical(u, 14) & (_NBUCK - 1)


def _atan2(y, x):
    ay = jnp.abs(y)
    ax = jnp.abs(x)
    hi = jnp.maximum(ay, ax)
    lo = jnp.minimum(ay, ax)
    t = lo / jnp.maximum(hi, 1e-30)
    s2 = t * t
    p = jnp.full(t.shape, _ATAN_C[-1], jnp.float32)
    for c in reversed(_ATAN_C[:-1]):
        p = p * s2 + c
    a = t * p
    a = jnp.where(ay > ax, _PI_2 - a, a)
    a = jnp.where(x < 0.0, _PI - a, a)
    a = jnp.where(y < 0.0, -a, a)
    return a


def _sc_body(sup, regf, heif, dimf, rotf, velf, out,
             data, hist, cand_v, cand_i, bhist, totals, mval, midx,
             cnts8, cntbuf, ch, rankb, rowbuf, zf, zi, zout,
             sh_hist, sh_cnt, sh_mval, sh_midx, sem):
    cax = lax.axis_index("c")
    sax = lax.axis_index("s")
    wid = cax * 16 + sax
    b = 2 * cax + sax // _NSLICE
    b2 = sax // _NSLICE
    j = sax % _NSLICE
    lane = lax.iota(jnp.int32, 16)
    z16i = jnp.zeros((16,), jnp.int32)
    z16f = jnp.zeros((16,), jnp.float32)

    base = b * _CHW + j * _SLICE
    dcp = pltpu.make_async_copy(sup.at[pl.ds(base, _SLICE)], data, sem)
    dcp.start()

    def _zf(i, _):
        zf[pl.ds(i * 16, 16)] = z16f
        zi[pl.ds(i * 16, 16)] = z16i
        return 0
    lax.fori_loop(0, 32, _zf, 0)

    def _zh(i, _):
        hist[pl.ds(i * 16, 16)] = z16i
        return 0
    lax.fori_loop(0, _NBUCK * 16 // 16, _zh, 0)

    def _zo(i, _):
        zout[i] = z16f
        rowbuf[i % 16] = z16f
        return 0
    lax.fori_loop(0, 64, _zo, 0)

    pltpu.sync_copy(zf, sh_mval.at[b2, pl.ds(j * 512, 512)])
    pltpu.sync_copy(zi, sh_midx.at[b2, pl.ds(j * 512, 512)])
    pltpu.sync_copy(zout, out.at[pl.ds(wid * 64, 64)])

    dcp.wait()

    ones16 = jnp.ones((16,), jnp.int32)

    def _ha(i, _):
        v = data[pl.ds(i * 16, 16)]
        bkt = _bucket_of(v)
        plsc.addupdate_scatter(hist, [lane * _NBUCK + bkt], ones16,
                               mask=v > 0.0)
        return 0
    lax.fori_loop(0, _SLICE // 16, _ha, 0)

    def _hb(i, acc_):
        def _hl(l, a):
            return a + hist[pl.ds(l * _NBUCK + i * 16, 16)]
        acc = lax.fori_loop(1, 16, _hl, hist[pl.ds(i * 16, 16)])
        totals[pl.ds(i * 16, 16)] = acc
        return 0
    lax.fori_loop(0, _NBUCK // 16, _hb, 0)
    pltpu.sync_copy(totals, sh_hist.at[b2, j])
    plsc.subcore_barrier()

    pltpu.sync_copy(sh_hist.at[b2], bhist)

    def _hc(i, _):
        def _hj(l, a):
            return a + bhist[l, pl.ds(i * 16, 16)]
        acc = lax.fori_loop(1, 8, _hj, bhist[0, pl.ds(i * 16, 16)])
        totals[pl.ds(i * 16, 16)] = acc
        return 0
    lax.fori_loop(0, _NBUCK // 16, _hc, 0)

    def _sb(i, carry):
        beta_, csum_ = carry
        cs = plsc.cumsum(totals[pl.ds(i * 16, 16)]) + csum_
        hit = cs >= _K
        anyh = jnp.any(hit)
        f = jnp.max(plsc.all_reduce_ffs(hit))
        cand_b = i * 16 + f
        beta_ = jnp.where((beta_ >= _NBUCK) & anyh, cand_b, beta_)
        return beta_, jnp.max(cs)
    beta, _tot = lax.fori_loop(0, _NBUCK // 16, _sb,
                               (jnp.int32(_NBUCK + 1), jnp.int32(0)))

    def _cd(i, cnt):
        v = data[pl.ds(i * 16, 16)]
        bkt = _bucket_of(v)
        m = (v > 0.0) & (bkt <= beta)
        nm = jnp.sum(m.astype(jnp.int32))
        ok = cnt < _CAND_CAP - 32

        @pl.when((nm > 0) & ok)
        def _():
            pos = jnp.full((16,), cnt, jnp.int32) + \
                plsc.cumsum(m.astype(jnp.int32)) - 1
            gidx = jnp.full((16,), j * _SLICE + i * 16, jnp.int32) + lane
            plsc.store_scatter(cand_v, [pos], v, mask=m)
            plsc.store_scatter(cand_i, [pos], gidx, mask=m)
        return cnt + jnp.where(ok, nm, 0)
    cnt = lax.fori_loop(0, _SLICE // 16, _cd, jnp.int32(0))

    cnt16 = (cnt + 15) // 16
    padm = lane < (cnt16 * 16 - cnt)
    pos = jnp.full((16,), cnt, jnp.int32) + lane
    plsc.store_scatter(cand_v, [pos], z16f, mask=padm)
    plsc.store_scatter(cand_i, [pos], z16i, mask=padm)

    cntbuf[pl.ds(0, 16)] = jnp.full((16,), cnt16, jnp.int32)
    pltpu.sync_copy(cntbuf, sh_cnt.at[b2, j])
    plsc.subcore_barrier()

    pltpu.sync_copy(sh_cnt.at[b2], cnts8)

    def _eb(jj, carry):
        base_, tot_ = carry
        cjj = cnts8[jj, pl.ds(0, 16)][0]
        return base_ + jnp.where(jj < j, cjj, 0), tot_ + cjj
    base16, total16 = lax.fori_loop(0, _NSLICE, _eb,
                                    (jnp.int32(0), jnp.int32(0)))

    def _ec(t, _):
        off = (base16 + t) * 16

        @pl.when(off <= _MERGE_CAP - 16)
        def _():
            pltpu.sync_copy(cand_v.at[pl.ds(t * 16, 16)],
                            sh_mval.at[b2, pl.ds(off, 16)])
            pltpu.sync_copy(cand_i.at[pl.ds(t * 16, 16)],
                            sh_midx.at[b2, pl.ds(off, 16)])
        return 0
    lax.fori_loop(0, cnt16, _ec, 0)
    plsc.subcore_barrier()

    pltpu.sync_copy(sh_mval.at[b2], mval)
    pltpu.sync_copy(sh_midx.at[b2], midx)
    total16c = jnp.minimum(total16, _MERGE_CAP // 16)

    def _gg(t, _):
        gi = cand_i[pl.ds(t * 16, 16)]
        pix = gi & (_HW - 1)
        d = []
        for k, (ref, nch, kk) in enumerate((
                (regf, 2, 0), (regf, 2, 1), (heif, 1, 0),
                (dimf, 3, 0), (dimf, 3, 1), (dimf, 3, 2),
                (rotf, 2, 0), (rotf, 2, 1),
                (velf, 2, 0), (velf, 2, 1))):
            a = b * (nch * _HW) + kk * _HW + pix
            d.append(pltpu.make_async_copy(
                ref.at[a], ch.at[k, pl.ds(t * 16, 16)], sem))
        for cp in d:
            cp.start()
        for cp in d:
            cp.wait()
        return 0
    lax.fori_loop(0, cnt16, _gg, 0)

    def _rk(i, _):
        i16 = jnp.full((16,), i, jnp.int32)
        vi = plsc.load_gather(cand_v, [i16])
        xi = plsc.load_gather(cand_i, [i16])

        def _rr(tt, acc):
            vj = mval[pl.ds(tt * 16, 16)]
            xj = midx[pl.ds(tt * 16, 16)]
            w = (vj > vi) | ((vj == vi) & (xj < xi))
            return acc + w.astype(jnp.int32)
        acc = lax.fori_loop(0, total16c, _rr, z16i)
        plsc.store_scatter(rankb, [i16],
                           jnp.full((16,), jnp.sum(acc), jnp.int32),
                           mask=lane == 0)
        return 0
    lax.fori_loop(0, cnt16 * 16, _rk, 0)

    def _oo(t, _):
        gi = cand_i[pl.ds(t * 16, 16)]
        sc = cand_v[pl.ds(t * 16, 16)]
        rk = rankb[pl.ds(t * 16, 16)]
        pix = gi & (_HW - 1)
        clsf = lax.shift_right_logical(gi, 18).astype(jnp.float32)
        ys = lax.shift_right_logical(pix, 9).astype(jnp.float32)
        xs = (pix & (_W - 1)).astype(jnp.float32)
        r0 = ch[0, pl.ds(t * 16, 16)]
        r1 = ch[1, pl.ds(t * 16, 16)]
        hei = ch[2, pl.ds(t * 16, 16)]
        e0 = jnp.exp(ch[3, pl.ds(t * 16, 16)])
        e1 = jnp.exp(ch[4, pl.ds(t * 16, 16)])
        e2 = jnp.exp(ch[5, pl.ds(t * 16, 16)])
        ang = _atan2(ch[6, pl.ds(t * 16, 16)], ch[7, pl.ds(t * 16, 16)])
        v0 = ch[8, pl.ds(t * 16, 16)]
        v1 = ch[9, pl.ds(t * 16, 16)]
        x = (xs + r0) * _OUT_SIZE_FACTOR
        y = (ys + r1) * _OUT_SIZE_FACTOR
        m = (sc > _SCORE_THRESHOLD) & (x > 0.0) & (x < _GRIDB) \
            & (y > 0.0) & (y < _GRIDB)
        scm = jnp.where(m, sc, 0.0)
        for k, val in enumerate((x, y, hei, e0, e1, e2, ang, v0, v1,
                                 scm, clsf)):
            plsc.store_scatter(rowbuf,
                               [lane, jnp.full((16,), k, jnp.int32)], val)
        rowi = jnp.where(rk < _K, b * _OROWS + rk,
                         b * _OROWS + _K + (lane & 7))
        ocp = pltpu.make_async_copy(rowbuf, out.at[rowi], sem)
        ocp.start()
        ocp.wait()
        return 0
    lax.fori_loop(0, cnt16, _oo, 0)


def _decode_sc(sup_flat, regf, heif, dimf, rotf, velf):
    mesh = plsc.VectorSubcoreMesh(core_axis_name="c", subcore_axis_name="s")
    fn = functools.partial(
        pl.kernel,
        mesh=mesh,
        compiler_params=pltpu.CompilerParams(needs_layout_passes=False,
                                             use_tc_tiling_on_sc=False),
        out_type=jax.ShapeDtypeStruct((_B * _OROWS, 16), jnp.float32),
        scratch_types=[
            pltpu.VMEM((_SLICE,), jnp.float32),
            pltpu.VMEM((16 * _NBUCK,), jnp.int32),
            pltpu.VMEM((_CAND_CAP,), jnp.float32),
            pltpu.VMEM((_CAND_CAP,), jnp.int32),
            pltpu.VMEM((_NSLICE, _NBUCK), jnp.int32),
            pltpu.VMEM((_NBUCK,), jnp.int32),
            pltpu.VMEM((_MERGE_CAP,), jnp.float32),
            pltpu.VMEM((_MERGE_CAP,), jnp.int32),
            pltpu.VMEM((_NSLICE, 16), jnp.int32),
            pltpu.VMEM((16,), jnp.int32),
            pltpu.VMEM((10, _CAND_CAP), jnp.float32),
            pltpu.VMEM((_CAND_CAP,), jnp.int32),
            pltpu.VMEM((16, 16), jnp.float32),
            pltpu.VMEM((512,), jnp.float32),
            pltpu.VMEM((512,), jnp.int32),
            pltpu.VMEM((64, 16), jnp.float32),
            pltpu.VMEM_SHARED((2, _NSLICE, _NBUCK), jnp.int32),
            pltpu.VMEM_SHARED((2, _NSLICE, 16), jnp.int32),
            pltpu.VMEM_SHARED((2, _MERGE_CAP), jnp.float32),
            pltpu.VMEM_SHARED((2, _MERGE_CAP), jnp.int32),
            pltpu.SemaphoreType.DMA,
        ],
    )(_sc_body)
    return fn(sup_flat, regf, heif, dimf, rotf, velf)


def kernel(heatmap, reg, height, dim, rot, vel):
    sup = _suppress(heatmap)
    res = _decode_sc(sup, reg.reshape(-1), height.reshape(-1),
                     dim.reshape(-1), rot.reshape(-1), vel.reshape(-1))
    return res.reshape(_B, _OROWS, 16)[:, :_K, :11]


# speedup vs baseline: 43.6185x; 1.7331x over previous
import functools

import jax
import jax.numpy as jnp
from jax import lax
from jax.experimental import pallas as pl
from jax.experimental.pallas import tpu as pltpu
from jax.experimental.pallas import tpu_sc as plsc

_K = 500
_OUT_SIZE_FACTOR = 4.0
_SCORE_THRESHOLD = 0.1
_GRIDB = 2048.0

_B, _C, _H, _W = 4, 2, 512, 512
_HW = _H * _W
_CHW = _C * _HW
_NSLICE = 8
_SLICE = _CHW // _NSLICE
_NBUCK = 512
_CAND_CAP = 2048
_MERGE_CAP = 4096
_OROWS = 512

_PI = 3.14159265358979
_PI_2 = 1.5707963267948966

_ATAN_C = (0.9999999981419218, -0.33333292787705715, 0.19998532263347163,
           -0.14264888583256646, 0.10958341227667072, -0.08427560725997432,
           0.05845650556360228, -0.0317490822377156, 0.011256772475624163,
           -0.001877352082647006)


def _suppress_body(x_ref, o_ref):
    x = x_ref[0]
    s = 1.0 / (1.0 + jnp.exp(-x))
    ninf = jnp.full((1, s.shape[1]), -jnp.inf, s.dtype)
    up = jnp.concatenate([s[1:], ninf], axis=0)
    dn = jnp.concatenate([ninf, s[:-1]], axis=0)
    r = jnp.maximum(jnp.maximum(s, up), dn)
    ninfc = jnp.full((r.shape[0], 1), -jnp.inf, r.dtype)
    lt = jnp.concatenate([r[:, 1:], ninfc], axis=1)
    rt = jnp.concatenate([ninfc, r[:, :-1]], axis=1)
    m = jnp.maximum(jnp.maximum(r, lt), rt)
    o_ref[0] = jnp.where(s == m, s, 0.0)


def _suppress(heatmap):
    B, C, H, W = heatmap.shape
    hm = heatmap.reshape(B * C, H, W)
    out = pl.pallas_call(
        _suppress_body,
        grid=(B * C,),
        in_specs=[pl.BlockSpec((1, H, W), lambda i: (i, 0, 0))],
        out_specs=pl.BlockSpec((1, H, W), lambda i: (i, 0, 0)),
        out_shape=jax.ShapeDtypeStruct((B * C, H, W), jnp.float32),
    )(hm)
    return out.reshape(B * C * H * W)


def _bucket_of(v):
    t = 2.0 - jnp.maximum(v, 1e-6)
    u = lax.bitcast_convert_type(t, jnp.int32)
    return lax.shift_right_logical(u, 14) & (_NBUCK - 1)


def _atan2(y, x):
    ay = jnp.abs(y)
    ax = jnp.abs(x)
    hi = jnp.maximum(ay, ax)
    lo = jnp.minimum(ay, ax)
    t = lo / jnp.maximum(hi, 1e-30)
    s2 = t * t
    p = jnp.full(t.shape, _ATAN_C[-1], jnp.float32)
    for c in reversed(_ATAN_C[:-1]):
        p = p * s2 + c
    a = t * p
    a = jnp.where(ay > ax, _PI_2 - a, a)
    a = jnp.where(x < 0.0, _PI - a, a)
    a = jnp.where(y < 0.0, -a, a)
    return a


def _sc_body(sup, regf, heif, dimf, rotf, velf, out,
             data, hist, cmax, cand_v, cand_i, bhist, totals, mval, midx,
             cnts8, cntbuf, ch, rankb, rowbuf, zf, zi, zout,
             sh_hist, sh_cnt, sh_mval, sh_midx, sem):
    cax = lax.axis_index("c")
    sax = lax.axis_index("s")
    wid = cax * 16 + sax
    b = 2 * cax + sax // _NSLICE
    b2 = sax // _NSLICE
    j = sax % _NSLICE
    lane = lax.iota(jnp.int32, 16)
    z16i = jnp.zeros((16,), jnp.int32)
    z16f = jnp.zeros((16,), jnp.float32)

    base = b * _CHW + j * _SLICE
    dcp = pltpu.make_async_copy(sup.at[pl.ds(base, _SLICE)], data, sem)
    dcp.start()

    def _zf(i, _):
        zf[pl.ds(i * 16, 16)] = z16f
        zi[pl.ds(i * 16, 16)] = z16i
        return 0
    lax.fori_loop(0, 32, _zf, 0)

    def _zh(i, _):
        for k in range(8):
            hist[pl.ds((i * 8 + k) * 16, 16)] = z16i
        return 0
    lax.fori_loop(0, _NBUCK * 16 // 128, _zh, 0)

    def _zo(i, _):
        zout[i] = z16f
        rowbuf[i % 16] = z16f
        return 0
    lax.fori_loop(0, 64, _zo, 0)

    pltpu.sync_copy(zf, sh_mval.at[b2, pl.ds(j * 512, 512)])
    pltpu.sync_copy(zi, sh_midx.at[b2, pl.ds(j * 512, 512)])
    pltpu.sync_copy(zout, out.at[pl.ds(wid * 64, 64)])

    dcp.wait()

    ones16 = jnp.ones((16,), jnp.int32)
    lane_nb = lane * _NBUCK

    def _ha(i, _):
        vs = []
        for k in range(4):
            v = data[pl.ds((i * 4 + k) * 16, 16)]
            vs.append(v)
            bkt = _bucket_of(v)
            plsc.addupdate_scatter(hist, [lane_nb + bkt], ones16,
                                   mask=v > 0.0)
        cm = jnp.max(jnp.maximum(jnp.maximum(vs[0], vs[1]),
                                 jnp.maximum(vs[2], vs[3])))
        plsc.store_scatter(cmax, [jnp.full((16,), i, jnp.int32)],
                           jnp.full((16,), cm, jnp.float32),
                           mask=lane == 0)
        return 0
    lax.fori_loop(0, _SLICE // 64, _ha, 0)

    def _hb(i, acc_):
        def _hl(l, a):
            return a + hist[pl.ds(l * _NBUCK + i * 16, 16)]
        acc = lax.fori_loop(1, 16, _hl, hist[pl.ds(i * 16, 16)])
        totals[pl.ds(i * 16, 16)] = acc
        return 0
    lax.fori_loop(0, _NBUCK // 16, _hb, 0)
    pltpu.sync_copy(totals, sh_hist.at[b2, j])
    plsc.subcore_barrier()

    pltpu.sync_copy(sh_hist.at[b2], bhist)

    def _hc(i, _):
        def _hj(l, a):
            return a + bhist[l, pl.ds(i * 16, 16)]
        acc = lax.fori_loop(1, 8, _hj, bhist[0, pl.ds(i * 16, 16)])
        totals[pl.ds(i * 16, 16)] = acc
        return 0
    lax.fori_loop(0, _NBUCK // 16, _hc, 0)

    def _sb(i, carry):
        beta_, csum_ = carry
        cs = plsc.cumsum(totals[pl.ds(i * 16, 16)]) + csum_
        hit = cs >= _K
        anyh = jnp.any(hit)
        f = jnp.max(plsc.all_reduce_ffs(hit))
        cand_b = i * 16 + f
        beta_ = jnp.where((beta_ >= _NBUCK) & anyh, cand_b, beta_)
        return beta_, jnp.max(cs)
    beta, _tot = lax.fori_loop(0, _NBUCK // 16, _sb,
                               (jnp.int32(_NBUCK + 1), jnp.int32(0)))

    def _blk(blk, c0):
        c2 = c0
        for q in range(4):
            i = blk * 4 + q
            v = data[pl.ds(i * 16, 16)]
            bkt = _bucket_of(v)
            mm = (v > 0.0) & (bkt <= beta)
            nm = jnp.sum(mm.astype(jnp.int32))
            ok = c2 < _CAND_CAP - 32

            @pl.when((nm > 0) & ok)
            def _():
                pos = jnp.full((16,), c2, jnp.int32) + \
                    plsc.cumsum(mm.astype(jnp.int32)) - 1
                gidx = jnp.full((16,), j * _SLICE + i * 16,
                                jnp.int32) + lane
                plsc.store_scatter(cand_v, [pos], v, mask=mm)
                plsc.store_scatter(cand_i, [pos], gidx, mask=mm)
            c2 = c2 + jnp.where(ok, nm, 0)
        return c2

    def _cd(g, cnt):
        cmv = cmax[pl.ds(g * 16, 16)]
        qm = (cmv > 0.0) & (_bucket_of(cmv) <= beta)

        def _qc(st):
            m_, _c = st
            return jnp.any(m_)

        def _qb(st):
            m_, c_ = st
            k = jnp.max(plsc.all_reduce_ffs(m_))
            c_ = _blk(g * 16 + k, c_)
            return m_ & (lane != k), c_
        _, cnt = lax.while_loop(_qc, _qb, (qm, cnt))
        return cnt
    cnt = lax.fori_loop(0, _SLICE // 64 // 16, _cd, jnp.int32(0))

    cnt16 = (cnt + 15) // 16
    padm = lane < (cnt16 * 16 - cnt)
    pos = jnp.full((16,), cnt, jnp.int32) + lane
    plsc.store_scatter(cand_v, [pos], z16f, mask=padm)
    plsc.store_scatter(cand_i, [pos], z16i, mask=padm)

    cntbuf[pl.ds(0, 16)] = jnp.full((16,), cnt16, jnp.int32)
    pltpu.sync_copy(cntbuf, sh_cnt.at[b2, j])
    plsc.subcore_barrier()

    pltpu.sync_copy(sh_cnt.at[b2], cnts8)

    def _eb(jj, carry):
        base_, tot_ = carry
        cjj = cnts8[jj, pl.ds(0, 16)][0]
        return base_ + jnp.where(jj < j, cjj, 0), tot_ + cjj
    base16, total16 = lax.fori_loop(0, _NSLICE, _eb,
                                    (jnp.int32(0), jnp.int32(0)))

    def _ec(t, _):
        off = (base16 + t) * 16

        @pl.when(off <= _MERGE_CAP - 16)
        def _():
            pltpu.sync_copy(cand_v.at[pl.ds(t * 16, 16)],
                            sh_mval.at[b2, pl.ds(off, 16)])
            pltpu.sync_copy(cand_i.at[pl.ds(t * 16, 16)],
                            sh_midx.at[b2, pl.ds(off, 16)])
        return 0
    lax.fori_loop(0, cnt16, _ec, 0)
    plsc.subcore_barrier()

    pltpu.sync_copy(sh_mval.at[b2], mval)
    pltpu.sync_copy(sh_midx.at[b2], midx)
    total16c = jnp.minimum(total16, _MERGE_CAP // 16)

    chans = ((regf, 2, 0), (regf, 2, 1), (heif, 1, 0),
             (dimf, 3, 0), (dimf, 3, 1), (dimf, 3, 2),
             (rotf, 2, 0), (rotf, 2, 1),
             (velf, 2, 0), (velf, 2, 1))

    def _gf(t, _):
        gi = cand_i[pl.ds(t * 16, 16)]
        pix = gi & (_HW - 1)
        for k, (ref, nch, kk) in enumerate(chans):
            a = b * (nch * _HW) + kk * _HW + pix
            pltpu.make_async_copy(
                ref.at[a], ch.at[k, pl.ds(t * 16, 16)], sem).start()
        return 0
    lax.fori_loop(0, cnt16, _gf, 0)

    def _gd(t, _):
        for k, (ref, nch, kk) in enumerate(chans):
            pltpu.make_async_copy(
                ref.at[lane], ch.at[k, pl.ds(t * 16, 16)], sem).wait()
        return 0
    lax.fori_loop(0, cnt16, _gd, 0)

    def _rk(i, _):
        i16 = jnp.full((16,), i, jnp.int32)
        vi = plsc.load_gather(cand_v, [i16])
        xi = plsc.load_gather(cand_i, [i16])

        def _rr(tt, acc):
            for q in range(4):
                vj = mval[pl.ds((tt * 4 + q) * 16, 16)]
                xj = midx[pl.ds((tt * 4 + q) * 16, 16)]
                w = (vj > vi) | ((vj == vi) & (xj < xi))
                acc = acc + w.astype(jnp.int32)
            return acc
        acc = lax.fori_loop(0, (total16c + 3) // 4, _rr, z16i)
        plsc.store_scatter(rankb, [i16],
                           jnp.full((16,), jnp.sum(acc), jnp.int32),
                           mask=lane == 0)
        return 0
    lax.fori_loop(0, cnt16 * 16, _rk, 0)

    def _oo(t, _):
        gi = cand_i[pl.ds(t * 16, 16)]
        sc = cand_v[pl.ds(t * 16, 16)]
        rk = rankb[pl.ds(t * 16, 16)]
        pix = gi & (_HW - 1)
        clsf = lax.shift_right_logical(gi, 18).astype(jnp.float32)
        ys = lax.shift_right_logical(pix, 9).astype(jnp.float32)
        xs = (pix & (_W - 1)).astype(jnp.float32)
        r0 = ch[0, pl.ds(t * 16, 16)]
        r1 = ch[1, pl.ds(t * 16, 16)]
        hei = ch[2, pl.ds(t * 16, 16)]
        e0 = jnp.exp(ch[3, pl.ds(t * 16, 16)])
        e1 = jnp.exp(ch[4, pl.ds(t * 16, 16)])
        e2 = jnp.exp(ch[5, pl.ds(t * 16, 16)])
        ang = _atan2(ch[6, pl.ds(t * 16, 16)], ch[7, pl.ds(t * 16, 16)])
        v0 = ch[8, pl.ds(t * 16, 16)]
        v1 = ch[9, pl.ds(t * 16, 16)]
        x = (xs + r0) * _OUT_SIZE_FACTOR
        y = (ys + r1) * _OUT_SIZE_FACTOR
        m = (sc > _SCORE_THRESHOLD) & (x > 0.0) & (x < _GRIDB) \
            & (y > 0.0) & (y < _GRIDB)
        scm = jnp.where(m, sc, 0.0)
        for k, val in enumerate((x, y, hei, e0, e1, e2, ang, v0, v1,
                                 scm, clsf)):
            plsc.store_scatter(rowbuf,
                               [lane, jnp.full((16,), k, jnp.int32)], val)
        rowi = jnp.where(rk < _K, b * _OROWS + rk,
                         b * _OROWS + _K + (lane & 7))
        ocp = pltpu.make_async_copy(rowbuf, out.at[rowi], sem)
        ocp.start()
        ocp.wait()
        return 0
    lax.fori_loop(0, cnt16, _oo, 0)


def _decode_sc(sup_flat, regf, heif, dimf, rotf, velf):
    mesh = plsc.VectorSubcoreMesh(core_axis_name="c", subcore_axis_name="s")
    fn = functools.partial(
        pl.kernel,
        mesh=mesh,
        compiler_params=pltpu.CompilerParams(needs_layout_passes=False,
                                             use_tc_tiling_on_sc=False),
        out_type=jax.ShapeDtypeStruct((_B * _OROWS, 16), jnp.float32),
        scratch_types=[
            pltpu.VMEM((_SLICE,), jnp.float32),
            pltpu.VMEM((16 * _NBUCK,), jnp.int32),
            pltpu.VMEM((_SLICE // 64,), jnp.float32),
            pltpu.VMEM((_CAND_CAP,), jnp.float32),
            pltpu.VMEM((_CAND_CAP,), jnp.int32),
            pltpu.VMEM((_NSLICE, _NBUCK), jnp.int32),
            pltpu.VMEM((_NBUCK,), jnp.int32),
            pltpu.VMEM((_MERGE_CAP,), jnp.float32),
            pltpu.VMEM((_MERGE_CAP,), jnp.int32),
            pltpu.VMEM((_NSLICE, 16), jnp.int32),
            pltpu.VMEM((16,), jnp.int32),
            pltpu.VMEM((10, _CAND_CAP), jnp.float32),
            pltpu.VMEM((_CAND_CAP,), jnp.int32),
            pltpu.VMEM((16, 16), jnp.float32),
            pltpu.VMEM((512,), jnp.float32),
            pltpu.VMEM((512,), jnp.int32),
            pltpu.VMEM((64, 16), jnp.float32),
            pltpu.VMEM_SHARED((2, _NSLICE, _NBUCK), jnp.int32),
            pltpu.VMEM_SHARED((2, _NSLICE, 16), jnp.int32),
            pltpu.VMEM_SHARED((2, _MERGE_CAP), jnp.float32),
            pltpu.VMEM_SHARED((2, _MERGE_CAP), jnp.int32),
            pltpu.SemaphoreType.DMA,
        ],
    )(_sc_body)
    return fn(sup_flat, regf, heif, dimf, rotf, velf)


def kernel(heatmap, reg, height, dim, rot, vel):
    sup = _suppress(heatmap)
    res = _decode_sc(sup, reg.reshape(-1), height.reshape(-1),
                     dim.reshape(-1), rot.reshape(-1), vel.reshape(-1))
    return res.reshape(_B, _OROWS, 16)[:, :_K, :11]
